# Initial kernel scaffold; baseline (speedup 1.0000x reference)
import jax
import jax.numpy as jnp
from jax.experimental import pallas as pl


def kernel(xy, wh, obj, truth, biases):
    raise NotImplementedError("write your pallas kernel here")


# speedup vs baseline: 2.2724x; 2.2724x over previous
import jax
import jax.numpy as jnp
from jax import lax
from jax.experimental import pallas as pl
from jax.experimental.pallas import tpu as pltpu

_A = 5
_H = 26
_W = 26
_T = 30
_HW = _H * _W
_POS_THRESH = 0.6


def _body(xy_ref, wh_ref, obj_ref, truth_ref, bias_ref,
          txy_ref, twh_ref, tw_ref, tobj_ref, tnoobj_ref, tlabel_ref):
    f32 = jnp.float32
    tx = truth_ref[0, :, 0:1]
    ty = truth_ref[0, :, 1:2]
    tw = truth_ref[0, :, 2:3]
    th = truth_ref[0, :, 3:4]
    tcls = truth_ref[0, :, 4:5]
    valid = tw > 1e-6

    twc = tw * _W
    thc = th * _H
    ci = jnp.clip((tx * _W).astype(jnp.int32), 0, _W - 1)
    cj = jnp.clip((ty * _H).astype(jnp.int32), 0, _H - 1)
    cellpos = cj * _W + ci
    tgt_x = tx * _W - ci.astype(f32)
    tgt_y = ty * _H - cj.astype(f32)
    wgt = 2.0 - tw * th

    best_r = jnp.full_like(tx, -1.0)
    ba = jnp.zeros_like(ci)
    bw_sel = jnp.zeros_like(tx)
    bh_sel = jnp.zeros_like(tx)
    for a in range(_A):
        bw_a = bias_ref[0:1, 2 * a:2 * a + 1]
        bh_a = bias_ref[0:1, 2 * a + 1:2 * a + 2]
        inter = jnp.minimum(twc, bw_a) * jnp.minimum(thc, bh_a)
        union = twc * thc + bw_a * bh_a - inter
        r = inter / jnp.maximum(union, 1e-12)
        upd = r > best_r
        best_r = jnp.where(upd, r, best_r)
        ba = jnp.where(upd, a, ba)
        bw_sel = jnp.where(upd, bw_a, bw_sel)
        bh_sel = jnp.where(upd, bh_a, bh_sel)
    tgt_w = jnp.log(jnp.maximum(twc, 1e-12) / bw_sel)
    tgt_h = jnp.log(jnp.maximum(thc, 1e-12) / bh_sel)

    tcol = lax.broadcasted_iota(jnp.int32, (_T, 1), 0)
    idx_row = lax.broadcasted_iota(jnp.int32, (1, _HW), 1)
    gx = (idx_row % _W).astype(f32)
    gy = (idx_row // _W).astype(f32)

    half_tw = tw * 0.5
    half_th = th * 0.5
    tl = tx - half_tw
    tr = tx + half_tw
    tt = ty - half_th
    tb = ty + half_th
    t_area = tw * th

    for a in range(_A):
        bw_a = bias_ref[0:1, 2 * a:2 * a + 1]
        bh_a = bias_ref[0:1, 2 * a + 1:2 * a + 2]
        xy0 = xy_ref[0, 2 * a:2 * a + 1, :]
        xy1 = xy_ref[0, 2 * a + 1:2 * a + 2, :]
        wh0 = wh_ref[0, 2 * a:2 * a + 1, :]
        wh1 = wh_ref[0, 2 * a + 1:2 * a + 2, :]
        obj_a = obj_ref[0, a:a + 1, :]

        px = (gx + xy0) * (1.0 / _W)
        py = (gy + xy1) * (1.0 / _H)
        pw = jnp.exp(wh0) * (bw_a * (1.0 / _W))
        ph = jnp.exp(wh1) * (bh_a * (1.0 / _H))
        half_pw = pw * 0.5
        half_ph = ph * 0.5

        l = jnp.maximum(px - half_pw, tl)
        r = jnp.minimum(px + half_pw, tr)
        t = jnp.maximum(py - half_ph, tt)
        b = jnp.minimum(py + half_ph, tb)
        inter = jnp.maximum(r - l, 0.0) * jnp.maximum(b - t, 0.0)
        union = pw * ph + t_area - inter
        iou = jnp.where(union > 0, inter / jnp.maximum(union, 1e-12), 0.0)
        iou = jnp.where(valid, iou, 0.0)

        best_iou = jnp.max(iou, axis=0, keepdims=True)

        match = valid & (ba == a) & (cellpos == idx_row)
        selid = jnp.max(jnp.where(match, tcol + 1, 0), axis=0, keepdims=True)
        assigned = selid > 0
        winner = (match & ((tcol + 1) == selid)).astype(f32)

        def pick(col):
            return jnp.sum(winner * col, axis=0, keepdims=True)

        x_win = pick(tgt_x)
        y_win = pick(tgt_y)
        w_win = pick(tgt_w)
        h_win = pick(tgt_h)
        g_win = pick(wgt)
        c_win = pick(tcls)
        iou_win = jnp.sum(winner * iou, axis=0, keepdims=True)

        txy_ref[0, 2 * a:2 * a + 1, :] = jnp.where(assigned, x_win, xy0)
        txy_ref[0, 2 * a + 1:2 * a + 2, :] = jnp.where(assigned, y_win, xy1)
        twh_ref[0, 2 * a:2 * a + 1, :] = jnp.where(assigned, w_win, wh0)
        twh_ref[0, 2 * a + 1:2 * a + 2, :] = jnp.where(assigned, h_win, wh1)
        gv = jnp.where(assigned, g_win, 0.0)
        tw_ref[0, 2 * a:2 * a + 1, :] = gv
        tw_ref[0, 2 * a + 1:2 * a + 2, :] = gv
        tobj_ref[0, a:a + 1, :] = jnp.where(assigned, iou_win, obj_a)
        tnoobj_ref[0, a:a + 1, :] = jnp.where(
            assigned | (best_iou > _POS_THRESH), obj_a, 0.0)
        tlabel_ref[0, a:a + 1, :] = jnp.where(assigned, c_win, -1.0)


def kernel(xy, wh, obj, truth, biases):
    B = xy.shape[0]
    xy_r = xy.reshape(B, 2 * _A, _HW)
    wh_r = wh.reshape(B, 2 * _A, _HW)
    obj_r = obj.reshape(B, _A, _HW)
    bias_r = biases.reshape(1, 2 * _A)

    out_shapes = (
        jax.ShapeDtypeStruct((B, 2 * _A, _HW), jnp.float32),
        jax.ShapeDtypeStruct((B, 2 * _A, _HW), jnp.float32),
        jax.ShapeDtypeStruct((B, 2 * _A, _HW), jnp.float32),
        jax.ShapeDtypeStruct((B, _A, _HW), jnp.float32),
        jax.ShapeDtypeStruct((B, _A, _HW), jnp.float32),
        jax.ShapeDtypeStruct((B, _A, _HW), jnp.float32),
    )
    in_specs = [
        pl.BlockSpec((1, 2 * _A, _HW), lambda b: (b, 0, 0)),
        pl.BlockSpec((1, 2 * _A, _HW), lambda b: (b, 0, 0)),
        pl.BlockSpec((1, _A, _HW), lambda b: (b, 0, 0)),
        pl.BlockSpec((1, _T, 5), lambda b: (b, 0, 0)),
        pl.BlockSpec((1, 2 * _A), lambda b: (0, 0)),
    ]
    out_specs = (
        pl.BlockSpec((1, 2 * _A, _HW), lambda b: (b, 0, 0)),
        pl.BlockSpec((1, 2 * _A, _HW), lambda b: (b, 0, 0)),
        pl.BlockSpec((1, 2 * _A, _HW), lambda b: (b, 0, 0)),
        pl.BlockSpec((1, _A, _HW), lambda b: (b, 0, 0)),
        pl.BlockSpec((1, _A, _HW), lambda b: (b, 0, 0)),
        pl.BlockSpec((1, _A, _HW), lambda b: (b, 0, 0)),
    )
    t_xy, t_wh, t_w, t_obj, t_noobj, t_label = pl.pallas_call(
        _body,
        grid=(B,),
        in_specs=in_specs,
        out_specs=out_specs,
        out_shape=out_shapes,
    )(xy_r, wh_r, obj_r, truth, bias_r)

    return (
        t_xy.reshape(B, 2 * _A, _H, _W),
        t_wh.reshape(B, 2 * _A, _H, _W),
        t_w.reshape(B, 2 * _A, _H, _W),
        t_obj.reshape(B, _A, _H, _W),
        t_noobj.reshape(B, _A, _H, _W),
        t_label.reshape(B, _A, _H, _W),
    )
